# Initial kernel scaffold; baseline (speedup 1.0000x reference)
import jax
import jax.numpy as jnp
from jax.experimental import pallas as pl


def kernel(position_ids, pe):
    raise NotImplementedError("write your pallas kernel here")


# speedup vs baseline: 4.9205x; 4.9205x over previous
import functools

import jax
import jax.numpy as jnp
from jax import lax
from jax.experimental import pallas as pl
from jax.experimental.pallas import tpu as pltpu
from jax.experimental.pallas import tpu_sc as plsc

_MAX_POSITION = 15000
_NUM_CORES = 2
_NUM_SUBCORES = 16
_NUM_WORKERS = _NUM_CORES * _NUM_SUBCORES
_CHUNK = 128
_LANES = 16


def kernel(position_ids, pe):
    batch, hist = position_ids.shape
    vocab, dim = pe.shape
    total = batch * hist
    per_worker = total // _NUM_WORKERS
    assert per_worker * _NUM_WORKERS == total
    nchunks = per_worker // _CHUNK
    assert nchunks * _CHUNK == per_worker and nchunks % 2 == 0
    npairs = nchunks // 2

    ids_flat = position_ids.reshape(total)
    mesh = plsc.VectorSubcoreMesh(core_axis_name="c", subcore_axis_name="s")

    @functools.partial(
        pl.kernel,
        mesh=mesh,
        out_type=jax.ShapeDtypeStruct((total, dim), jnp.float32),
        compiler_params=pltpu.CompilerParams(use_tc_tiling_on_sc=False),
        scratch_types=[
            pltpu.VMEM((per_worker,), jnp.int32),
            pltpu.VMEM((_CHUNK, dim), jnp.float32),
            pltpu.VMEM((_CHUNK, dim), jnp.float32),
            pltpu.SemaphoreType.DMA,
            pltpu.SemaphoreType.DMA,
            pltpu.SemaphoreType.DMA,
            pltpu.SemaphoreType.DMA,
        ],
    )
    def run(ids_hbm, pe_hbm, out_hbm, idx_v, rows0, rows1, gs0, gs1, ps0, ps1):
        wid = lax.axis_index("s") * _NUM_CORES + lax.axis_index("c")
        base = pl.multiple_of(wid * per_worker, _CHUNK)
        pltpu.sync_copy(ids_hbm.at[pl.ds(base, per_worker)], idx_v)

        def clamp(ch):
            off = ch * _CHUNK
            for j in range(_CHUNK // _LANES):
                sl = pl.ds(off + j * _LANES, _LANES)
                idx_v[sl] = jnp.minimum(idx_v[sl], _MAX_POSITION - 1)

        def start_gather(ch, rows, sem):
            idx_slice = idx_v.at[pl.ds(ch * _CHUNK, _CHUNK)]
            pltpu.make_async_copy(pe_hbm.at[idx_slice], rows, sem).start()

        def wait_gather(rows, sem):
            pltpu.make_async_copy(pe_hbm.at[pl.ds(0, _CHUNK)], rows, sem).wait()

        def start_put(ch, rows, sem):
            dst = out_hbm.at[pl.ds(base + ch * _CHUNK, _CHUNK)]
            pltpu.make_async_copy(rows, dst, sem).start()

        def wait_put(rows, sem):
            pltpu.make_async_copy(rows, out_hbm.at[pl.ds(base, _CHUNK)], sem).wait()

        clamp(0)
        clamp(1)
        start_gather(0, rows0, gs0)
        start_gather(1, rows1, gs1)

        def body(i, carry):
            c0 = 2 * i
            wait_gather(rows0, gs0)
            start_put(c0, rows0, ps0)
            wait_gather(rows1, gs1)
            start_put(c0 + 1, rows1, ps1)
            clamp(c0 + 2)
            wait_put(rows0, ps0)
            start_gather(c0 + 2, rows0, gs0)
            clamp(c0 + 3)
            wait_put(rows1, ps1)
            start_gather(c0 + 3, rows1, gs1)
            return carry

        lax.fori_loop(0, npairs - 1, body, None)

        last = nchunks - 2
        wait_gather(rows0, gs0)
        start_put(last, rows0, ps0)
        wait_gather(rows1, gs1)
        start_put(last + 1, rows1, ps1)
        wait_put(rows0, ps0)
        wait_put(rows1, ps1)

    out = run(ids_flat, pe)
    return out.reshape(batch, hist, dim)


# speedup vs baseline: 5.1770x; 1.0521x over previous
import functools

import jax
import jax.numpy as jnp
from jax import lax
from jax.experimental import pallas as pl
from jax.experimental.pallas import tpu as pltpu
from jax.experimental.pallas import tpu_sc as plsc

_MAX_POSITION = 15000
_NUM_CORES = 2
_NUM_SUBCORES = 16
_NUM_WORKERS = _NUM_CORES * _NUM_SUBCORES
_CHUNK = 128
_LANES = 16
_NSLOT = 8
_LOOKAHEAD = 4
_BLK = _NSLOT * _CHUNK


def kernel(position_ids, pe):
    batch, hist = position_ids.shape
    vocab, dim = pe.shape
    total = batch * hist
    per_worker = total // _NUM_WORKERS
    assert per_worker * _NUM_WORKERS == total
    nchunks = per_worker // _CHUNK
    ngroups = nchunks // _NSLOT
    assert ngroups * _NSLOT == nchunks and ngroups % 2 == 0 and ngroups >= 4

    ids_flat = position_ids.reshape(total)
    mesh = plsc.VectorSubcoreMesh(core_axis_name="c", subcore_axis_name="s")

    @functools.partial(
        pl.kernel,
        mesh=mesh,
        out_type=jax.ShapeDtypeStruct((total, dim), jnp.float32),
        compiler_params=pltpu.CompilerParams(use_tc_tiling_on_sc=False),
        scratch_types=[
            pltpu.VMEM((2, _BLK), jnp.int32),
            pltpu.VMEM((_NSLOT, _CHUNK, dim), jnp.float32),
            pltpu.SemaphoreType.DMA((2,)),
            pltpu.SemaphoreType.DMA((_NSLOT,)),
            pltpu.SemaphoreType.DMA((_NSLOT,)),
        ],
    )
    def run(ids_hbm, pe_hbm, out_hbm, iblk, rows, isem, gsem, psem):
        sid = lax.axis_index("s")
        wid = sid * _NUM_CORES + lax.axis_index("c")
        base = pl.multiple_of(wid * per_worker, _CHUNK)

        def load_block(u, slot):
            src = ids_hbm.at[pl.ds(base + u * _BLK, _BLK)]
            pltpu.make_async_copy(src, iblk.at[slot], isem.at[slot]).start()

        def wait_block(slot):
            src = ids_hbm.at[pl.ds(base, _BLK)]
            pltpu.make_async_copy(src, iblk.at[slot], isem.at[slot]).wait()

        def clamp(bslot, pos):
            for j in range(_CHUNK // _LANES):
                sl = pl.ds(pos * _CHUNK + j * _LANES, _LANES)
                iblk[bslot, sl] = jnp.minimum(iblk[bslot, sl], _MAX_POSITION - 1)

        def start_gather(c, slot, bslot, pos):
            idx_sl = iblk.at[bslot, pl.ds(pos * _CHUNK, _CHUNK)]
            pltpu.make_async_copy(pe_hbm.at[idx_sl], rows.at[slot], gsem.at[slot]).start()

        def wait_gather(slot):
            src = pe_hbm.at[pl.ds(0, _CHUNK)]
            pltpu.make_async_copy(src, rows.at[slot], gsem.at[slot]).wait()

        def start_put(c, slot):
            dst = out_hbm.at[pl.ds(base + c * _CHUNK, _CHUNK)]
            pltpu.make_async_copy(rows.at[slot], dst, psem.at[slot]).start()

        def wait_put(slot):
            dst = out_hbm.at[pl.ds(base, _CHUNK)]
            pltpu.make_async_copy(rows.at[slot], dst, psem.at[slot]).wait()

        def body(c, b, cur, do_wait_put=True, prefetch=True):
            wait_gather(b)
            start_put(c, b)
            if prefetch:
                ps = (b + _LOOKAHEAD) % _NSLOT
                bs = cur if b < _LOOKAHEAD else (1 - cur)
                pos = (b + _LOOKAHEAD) % _NSLOT
                if b == _LOOKAHEAD:
                    wait_block(1 - cur)
                clamp(bs, pos)
                if do_wait_put:
                    wait_put(ps)
                start_gather(c + _LOOKAHEAD, ps, bs, pos)

        load_block(0, 0)
        wait_block(0)
        for b in range(_LOOKAHEAD):
            clamp(0, b)
            start_gather(b, b, 0, b)

        load_block(1, 1)
        for b in range(_NSLOT):
            body(b, b, cur=0, do_wait_put=(b >= _LOOKAHEAD))

        def pair(i, carry):
            for k in range(2):
                u = 2 * i + 1 + k
                cur = (1 + k) % 2
                load_block(u + 1, 1 - cur)
                c0 = u * _NSLOT
                for b in range(_NSLOT):
                    body(c0 + b, b, cur=cur)
            return carry

        lax.fori_loop(0, (ngroups - 2) // 2, pair, None)

        c0 = (ngroups - 1) * _NSLOT
        last_cur = (ngroups - 1) % 2
        for b in range(_NSLOT):
            body(c0 + b, b, cur=last_cur, prefetch=(b < _LOOKAHEAD))

        for b in range(_NSLOT):
            wait_put(b)

    out = run(ids_flat, pe)
    return out.reshape(batch, hist, dim)


# speedup vs baseline: 5.8215x; 1.1245x over previous
import functools

import jax
import jax.numpy as jnp
from jax import lax
from jax.experimental import pallas as pl
from jax.experimental.pallas import tpu as pltpu
from jax.experimental.pallas import tpu_sc as plsc

_MAX_POSITION = 15000
_NUM_CORES = 2
_NUM_SUBCORES = 16
_NUM_WORKERS = _NUM_CORES * _NUM_SUBCORES
_CHUNK = 128
_LANES = 16
_NSLOT = 8
_LOOKAHEAD = 4
_BLK = _NSLOT * _CHUNK


def kernel(position_ids, pe):
    batch, hist = position_ids.shape
    vocab, dim = pe.shape
    total = batch * hist
    per_worker = total // _NUM_WORKERS
    assert per_worker * _NUM_WORKERS == total
    nchunks = per_worker // _CHUNK
    ngroups = nchunks // _NSLOT
    assert ngroups * _NSLOT == nchunks and ngroups % 2 == 0 and ngroups >= 4

    ids_flat = position_ids.reshape(total)
    mesh = plsc.VectorSubcoreMesh(core_axis_name="c", subcore_axis_name="s")

    @functools.partial(
        pl.kernel,
        mesh=mesh,
        out_type=jax.ShapeDtypeStruct((total, dim), jnp.float32),
        compiler_params=pltpu.CompilerParams(use_tc_tiling_on_sc=False),
        scratch_types=[
            pltpu.VMEM((2, _BLK), jnp.int32),
            pltpu.VMEM((_NSLOT, _CHUNK, dim), jnp.float32),
            pltpu.VMEM_SHARED((vocab, dim), jnp.float32),
            pltpu.SemaphoreType.DMA((2,)),
            pltpu.SemaphoreType.DMA((_NSLOT,)),
            pltpu.SemaphoreType.DMA((_NSLOT,)),
        ],
    )
    def run(ids_hbm, pe_hbm, out_hbm, iblk, rows, pe_sh, isem, gsem, psem):
        sid = lax.axis_index("s")
        wid = sid * _NUM_CORES + lax.axis_index("c")
        base = pl.multiple_of(wid * per_worker, _CHUNK)

        @pl.when(sid == 0)
        def _stage():
            pltpu.sync_copy(pe_hbm, pe_sh)

        plsc.subcore_barrier()

        def load_block(u, slot):
            src = ids_hbm.at[pl.ds(base + u * _BLK, _BLK)]
            pltpu.make_async_copy(src, iblk.at[slot], isem.at[slot]).start()

        def wait_block(slot):
            src = ids_hbm.at[pl.ds(base, _BLK)]
            pltpu.make_async_copy(src, iblk.at[slot], isem.at[slot]).wait()

        def clamp(bslot, pos):
            for j in range(_CHUNK // _LANES):
                sl = pl.ds(pos * _CHUNK + j * _LANES, _LANES)
                iblk[bslot, sl] = jnp.minimum(iblk[bslot, sl], _MAX_POSITION - 1)

        def start_gather(c, slot, bslot, pos):
            idx_sl = iblk.at[bslot, pl.ds(pos * _CHUNK, _CHUNK)]
            pltpu.make_async_copy(pe_sh.at[idx_sl], rows.at[slot], gsem.at[slot]).start()

        def wait_gather(slot):
            src = pe_hbm.at[pl.ds(0, _CHUNK)]
            pltpu.make_async_copy(src, rows.at[slot], gsem.at[slot]).wait()

        def start_put(c, slot):
            dst = out_hbm.at[pl.ds(base + c * _CHUNK, _CHUNK)]
            pltpu.make_async_copy(rows.at[slot], dst, psem.at[slot]).start()

        def wait_put(slot):
            dst = out_hbm.at[pl.ds(base, _CHUNK)]
            pltpu.make_async_copy(rows.at[slot], dst, psem.at[slot]).wait()

        def body(c, b, cur, do_wait_put=True, prefetch=True):
            wait_gather(b)
            start_put(c, b)
            if prefetch:
                ps = (b + _LOOKAHEAD) % _NSLOT
                bs = cur if b < _LOOKAHEAD else (1 - cur)
                pos = (b + _LOOKAHEAD) % _NSLOT
                if b == _LOOKAHEAD:
                    wait_block(1 - cur)
                clamp(bs, pos)
                if do_wait_put:
                    wait_put(ps)
                start_gather(c + _LOOKAHEAD, ps, bs, pos)

        load_block(0, 0)
        wait_block(0)
        for b in range(_LOOKAHEAD):
            clamp(0, b)
            start_gather(b, b, 0, b)

        load_block(1, 1)
        for b in range(_NSLOT):
            body(b, b, cur=0, do_wait_put=(b >= _LOOKAHEAD))

        def pair(i, carry):
            for k in range(2):
                u = 2 * i + 1 + k
                cur = (1 + k) % 2
                load_block(u + 1, 1 - cur)
                c0 = u * _NSLOT
                for b in range(_NSLOT):
                    body(c0 + b, b, cur=cur)
            return carry

        lax.fori_loop(0, (ngroups - 2) // 2, pair, None)

        c0 = (ngroups - 1) * _NSLOT
        last_cur = (ngroups - 1) % 2
        for b in range(_NSLOT):
            body(c0 + b, b, cur=last_cur, prefetch=(b < _LOOKAHEAD))

        for b in range(_NSLOT):
            wait_put(b)

    out = run(ids_flat, pe)
    return out.reshape(batch, hist, dim)
